# baseline (speedup 1.0000x reference)
import functools

import jax
import jax.numpy as jnp
from jax import lax
from jax.experimental import pallas as pl
from jax.experimental.pallas import tpu as pltpu
from jax.experimental.pallas import tpu_sc as plsc

VOCAB = 100000
SEQ_LEN = 200
BATCH = 1024
NTOK = SEQ_LEN * BATCH

HPAD = 51200
HSTART = (0, 51200)
TOKEN_WAYS = 16
VOCAB_WAYS = 2
BINS_PER_TILE = HPAD // VOCAB_WAYS
TOK_PER_TILE = NTOK // TOKEN_WAYS
VREGS_PER_TILE = TOK_PER_TILE // 16

BW = 2048
NCB = (25, 24)


def _sc_histogram_half(src_flat, half):
    mesh = plsc.VectorSubcoreMesh(core_axis_name="c", subcore_axis_name="s")
    h0 = HSTART[half]

    @functools.partial(
        pl.kernel,
        mesh=mesh,
        out_type=jax.ShapeDtypeStruct((TOKEN_WAYS * HPAD,), jnp.float32),
        compiler_params=pltpu.CompilerParams(needs_layout_passes=False),
        scratch_types=[
            pltpu.VMEM((TOK_PER_TILE,), jnp.int32),
            pltpu.VMEM((BINS_PER_TILE,), jnp.float32),
            pltpu.SemaphoreType.DMA,
        ],
    )
    def hist_kernel(src_hbm, out_hbm, buf, hist, sem):
        c = lax.axis_index("c")
        s = lax.axis_index("s")
        wid = s * 2 + c
        g = wid // VOCAB_WAYS
        v = wid % VOCAB_WAYS
        base = h0 + v * BINS_PER_TILE

        cp = pltpu.async_copy(
            src_hbm.at[pl.ds(g * TOK_PER_TILE, TOK_PER_TILE)], buf, sem)

        zeros16 = jnp.zeros((16,), jnp.float32)

        def zero_body(i, carry):
            hist[pl.ds(i * 16, 16)] = zeros16
            return carry

        lax.fori_loop(0, BINS_PER_TILE // 16, zero_body, 0)

        ones16 = jnp.ones((16,), jnp.float32)
        cp.wait()

        def body(i, carry):
            tok = buf[pl.ds(i * 16, 16)]
            rel = tok - base
            mask = (rel >= 0) & (rel < BINS_PER_TILE)
            plsc.addupdate_scatter(hist, [rel], ones16, mask=mask)
            return carry

        lax.fori_loop(0, VREGS_PER_TILE, body, 0)

        pltpu.sync_copy(
            hist,
            out_hbm.at[pl.ds(g * HPAD + v * BINS_PER_TILE, BINS_PER_TILE)])

    return hist_kernel(src_flat).reshape(TOKEN_WAYS, HPAD)


def _tc_broadcast_half(hist_parts, ign, half, prev=None):
    cb0 = half * NCB[0]

    def body(ign_ref, hist_ref, *rest):
        out_ref = rest[-1]
        jc = pl.program_id(0)
        summed = jnp.sum(hist_ref[...], axis=0, keepdims=True)
        cols = (cb0 + jc) * BW + lax.broadcasted_iota(
            jnp.int32, (1, BW), 1)
        ign_v = ign_ref[0]
        keep = (cols == ign_v) | (cols == 1) | (cols == 2)
        row = jnp.where(keep, 0.0, summed)
        out_ref[...] = jnp.broadcast_to(row, (BATCH, BW))

    in_specs = [
        pl.BlockSpec(memory_space=pltpu.SMEM),
        pl.BlockSpec((TOKEN_WAYS, BW), lambda jc: (0, jc)),
    ]
    args = [ign, hist_parts]
    aliases = {}
    if prev is not None:
        in_specs.append(pl.BlockSpec(memory_space=pl.ANY))
        args.append(prev)
        aliases = {2: 0}

    return pl.pallas_call(
        body,
        grid=(NCB[half],),
        in_specs=in_specs,
        out_specs=pl.BlockSpec((BATCH, BW), lambda jc, cb0=cb0: (0, cb0 + jc)),
        out_shape=jax.ShapeDtypeStruct((BATCH, VOCAB), jnp.float32),
        input_output_aliases=aliases,
    )(*args)


def kernel(src, ignore_index):
    src_flat = src.reshape(-1)
    ign = jnp.asarray(ignore_index, jnp.int32).reshape(1)
    parts0 = _sc_histogram_half(src_flat, 0)
    parts1 = _sc_histogram_half(src_flat, 1)
    out = _tc_broadcast_half(parts0, ign, 0)
    out = _tc_broadcast_half(parts1, ign, 1, prev=out)
    return out
